# baseline (speedup 1.0000x reference)
import functools

import jax
import jax.numpy as jnp
from jax import lax
from jax.experimental import pallas as pl
from jax.experimental.pallas import tpu as pltpu
from jax.experimental.pallas import tpu_sc as plsc

DIM = 32
NC, NS = 2, 16
NW = NC * NS
CHUNK = 128


@functools.cache
def _gather_fn(B):
    b_w = B // NW
    n_ch = b_w // CHUNK
    mesh = plsc.VectorSubcoreMesh(
        core_axis_name="c", subcore_axis_name="s",
        num_cores=NC, num_subcores=NS)

    def body(uidx_hbm, pidx_hbm, nidx_hbm, utab_hbm, itab_hbm,
             u_out, p_out, n_out,
             idx_v, rows_u, rows_p, rows_n, sem):
        wid = lax.axis_index("s") * NC + lax.axis_index("c")
        base = wid * b_w
        pltpu.sync_copy(uidx_hbm.at[wid], idx_v.at[0])
        pltpu.sync_copy(pidx_hbm.at[wid], idx_v.at[1])
        pltpu.sync_copy(nidx_hbm.at[wid], idx_v.at[2])
        copies = []
        for j in range(n_ch):
            copies.append(pltpu.async_copy(
                utab_hbm.at[idx_v.at[0, j]], rows_u.at[pl.ds(j * CHUNK, CHUNK)], sem))
            copies.append(pltpu.async_copy(
                itab_hbm.at[idx_v.at[1, j]], rows_p.at[pl.ds(j * CHUNK, CHUNK)], sem))
            copies.append(pltpu.async_copy(
                itab_hbm.at[idx_v.at[2, j]], rows_n.at[pl.ds(j * CHUNK, CHUNK)], sem))
        for c in copies:
            c.wait()
        pltpu.sync_copy(rows_u, u_out.at[pl.ds(base, b_w)])
        pltpu.sync_copy(rows_p, p_out.at[pl.ds(base, b_w)])
        pltpu.sync_copy(rows_n, n_out.at[pl.ds(base, b_w)])

    return pl.kernel(
        body,
        out_type=[jax.ShapeDtypeStruct((B, DIM), jnp.float32)] * 3,
        mesh=mesh,
        compiler_params=pltpu.CompilerParams(use_tc_tiling_on_sc=False),
        scratch_types=[
            pltpu.VMEM((3, n_ch, CHUNK), jnp.int32),
            pltpu.VMEM((b_w, DIM), jnp.float32),
            pltpu.VMEM((b_w, DIM), jnp.float32),
            pltpu.VMEM((b_w, DIM), jnp.float32),
            pltpu.SemaphoreType.DMA,
        ],
    )


def _sim_body(u_ref, p_ref, n_ref, o_ref):
    u = u_ref[...]
    p = p_ref[...]
    n = n_ref[...]
    eps = 1e-12
    uu = jnp.sum(u * u, axis=1)
    pp = jnp.sum(p * p, axis=1)
    nn = jnp.sum(n * n, axis=1)
    up = jnp.sum(u * p, axis=1)
    un = jnp.sum(u * n, axis=1)
    ru = lax.rsqrt(jnp.maximum(uu, eps))
    pos = up * ru * lax.rsqrt(jnp.maximum(pp, eps))
    neg = un * ru * lax.rsqrt(jnp.maximum(nn, eps))
    o_ref[...] = jnp.maximum(neg - pos + 1.0, 0.0)


@functools.cache
def _sim_fn(B):
    blk = 2048
    grid = B // blk
    return pl.pallas_call(
        _sim_body,
        grid=(grid,),
        in_specs=[pl.BlockSpec((blk, DIM), lambda i: (i, 0))] * 3,
        out_specs=pl.BlockSpec((blk,), lambda i: (i,)),
        out_shape=jax.ShapeDtypeStruct((B,), jnp.float32),
    )


def kernel(user_input, pos_item_input, neg_item_input, user_table, item_table):
    B = user_input.shape[0]
    ui = user_input.astype(jnp.int32).reshape(NW, -1, CHUNK)
    pi = pos_item_input.astype(jnp.int32).reshape(NW, -1, CHUNK)
    ni = neg_item_input.astype(jnp.int32).reshape(NW, -1, CHUNK)
    u, p, n = _gather_fn(B)(ui, pi, ni, user_table, item_table)
    return _sim_fn(B)(u, p, n).reshape(B, 1)
